# baseline (speedup 1.0000x reference)
import functools

import jax
import jax.numpy as jnp
from jax import lax
from jax.experimental import pallas as pl
from jax.experimental.pallas import tpu as pltpu
from jax.experimental.pallas import tpu_sc as plsc

N, C, H, W = 4, 128, 64, 64
K = 9
HW = H * W
CB = 16
CP = CB // 2
NWORKERS = 32
LPP = HW // 16
HALF = HW // 2


def _body(tgt_hbm, dcn_hbm, out_hbm, *refs):
    pplanes = refs[0:CP]
    outs = refs[CP:CP + CB]
    base = CP + CB
    offy = (refs[base], refs[base + 1])
    offx = (refs[base + 2], refs[base + 3])
    sem_pl = refs[base + 4]
    sem_off = (refs[base + 5], refs[base + 6])
    sem_out = (refs[base + 7], refs[base + 8])

    wid = lax.axis_index("s") * 2 + lax.axis_index("c")
    lane = lax.iota(jnp.int32, 16)

    n = wid // (C // CB)
    c0 = (wid % (C // CB)) * CB

    def compute_half(k, ob, half):
        ay = (k // 3 - 1).astype(jnp.float32)
        ax = (k % 3 - 1).astype(jnp.float32)
        offy_v = offy[ob]
        offx_v = offx[ob]

        @plsc.parallel_loop(half * (LPP // 2), (half + 1) * (LPP // 2),
                            unroll=1)
        def jbody(j):
            oy = offy_v[pl.ds(j * 16, 16)]
            ox = offx_v[pl.ds(j * 16, 16)]
            h = j // 4
            wb = (j % 4) * 16

            yv = oy + (h.astype(jnp.float32) + ay)
            xv = ox + (wb + lane).astype(jnp.float32) + ax

            ytf = yv.astype(jnp.int32).astype(jnp.float32)
            y0f = ytf - jnp.where(ytf > yv, 1.0, 0.0)
            xtf = xv.astype(jnp.int32).astype(jnp.float32)
            x0f = xtf - jnp.where(xtf > xv, 1.0, 0.0)

            wy1 = yv - y0f
            wy0 = 1.0 - wy1
            wx1 = xv - x0f
            wx0 = 1.0 - wx1
            y0c = jnp.minimum(jnp.maximum(y0f, 0.0), H - 1.0)
            x0c = jnp.minimum(jnp.maximum(x0f, 0.0), W - 1.0)
            wy0 = jnp.where(y0f == y0c, wy0, 0.0)
            wx0 = jnp.where(x0f == x0c, wx0, 0.0)
            wy1 = jnp.where((y0f >= -1.0) & (y0f <= H - 2.0), wy1, 0.0)
            wx1 = jnp.where((x0f >= -1.0) & (x0f <= W - 2.0), wx1, 0.0)

            y0i = y0c.astype(jnp.int32)
            x0i = x0c.astype(jnp.int32)
            y1i = jnp.minimum(jnp.maximum(y0f + 1.0, 0.0),
                              H - 1.0).astype(jnp.int32)
            x1i = jnp.minimum(jnp.maximum(x0f + 1.0, 0.0),
                              W - 1.0).astype(jnp.int32)

            iy0 = y0i * W
            iy1 = y1i * W
            i00 = iy0 + x0i
            i01 = iy0 + x1i
            i10 = iy1 + x0i
            i11 = iy1 + x1i

            wy0p = plsc.pack(wy0, wy0, format=plsc.PackFormat.INTERLEAVED)
            wy1p = plsc.pack(wy1, wy1, format=plsc.PackFormat.INTERLEAVED)
            wx0p = plsc.pack(wx0, wx0, format=plsc.PackFormat.INTERLEAVED)
            wx1p = plsc.pack(wx1, wx1, format=plsc.PackFormat.INTERLEAVED)

            for cp in range(CP):
                v00 = plsc.bitcast(
                    plsc.load_gather(pplanes[cp], [i00]), jnp.bfloat16)
                v01 = plsc.bitcast(
                    plsc.load_gather(pplanes[cp], [i01]), jnp.bfloat16)
                v10 = plsc.bitcast(
                    plsc.load_gather(pplanes[cp], [i10]), jnp.bfloat16)
                v11 = plsc.bitcast(
                    plsc.load_gather(pplanes[cp], [i11]), jnp.bfloat16)
                accp = (v00 * wx0p + v01 * wx1p) * wy0p \
                    + (v10 * wx0p + v11 * wx1p) * wy1p
                a0, a1 = plsc.unpack(accp, format=plsc.PackFormat.INTERLEAVED)
                outs[2 * cp][pl.ds(j * 16, 16)] = a0
                outs[2 * cp + 1][pl.ds(j * 16, 16)] = a1

    def issue_out(k, half):
        for ci in range(CB):
            pltpu.async_copy(
                outs[ci].at[pl.ds(half * HALF, HALF)],
                out_hbm.at[n, (c0 + ci) * K + k, half],
                sem_out[half])

    def drain_out(half):
        for ci in range(CB):
            pltpu.make_async_copy(
                outs[ci].at[pl.ds(half * HALF, HALF)],
                out_hbm.at[n, ci, half],
                sem_out[half]).wait()

    def prefetch_off(k, ob):
        pltpu.async_copy(dcn_hbm.at[n, 2 * k], offy[ob], sem_off[ob])
        pltpu.async_copy(dcn_hbm.at[n, 2 * k + 1], offx[ob], sem_off[ob])

    def drain_off(ob):
        pltpu.make_async_copy(dcn_hbm.at[n, 0], offy[ob], sem_off[ob]).wait()
        pltpu.make_async_copy(dcn_hbm.at[n, 0], offx[ob], sem_off[ob]).wait()

    for ci in range(CB):
        pltpu.async_copy(tgt_hbm.at[n, c0 + ci], outs[ci], sem_pl)
    prefetch_off(0, 0)
    for ci in range(CB):
        pltpu.make_async_copy(tgt_hbm.at[n, 0], outs[ci], sem_pl).wait()
    for cp in range(CP):
        ta = outs[2 * cp]
        tb = outs[2 * cp + 1]
        pp = pplanes[cp]

        @plsc.parallel_loop(0, LPP, unroll=2)
        def pack_body(j, ta=ta, tb=tb, pp=pp):
            va = ta[pl.ds(j * 16, 16)]
            vb = tb[pl.ds(j * 16, 16)]
            packed = plsc.pack(va, vb, format=plsc.PackFormat.INTERLEAVED)
            pp[pl.ds(j * 16, 16)] = plsc.bitcast(packed, jnp.int32)

    k0 = jnp.int32(0)
    drain_off(0)
    prefetch_off(1, 1)
    compute_half(k0, 0, 0)
    issue_out(k0, 0)
    compute_half(k0, 0, 1)
    issue_out(k0, 1)

    def pair_body(kk, carry):
        k1 = 2 * kk + 1
        drain_off(1)
        pltpu.async_copy(dcn_hbm.at[n, 2 * (k1 + 1)], offy[0], sem_off[0])
        pltpu.async_copy(dcn_hbm.at[n, 2 * (k1 + 1) + 1], offx[0], sem_off[0])
        drain_out(0)
        compute_half(k1, 1, 0)
        issue_out(k1, 0)
        drain_out(1)
        compute_half(k1, 1, 1)
        issue_out(k1, 1)

        k2 = k1 + 1
        drain_off(0)

        @pl.when(kk < 3)
        def _():
            pltpu.async_copy(dcn_hbm.at[n, 2 * (k2 + 1)], offy[1], sem_off[1])
            pltpu.async_copy(
                dcn_hbm.at[n, 2 * (k2 + 1) + 1], offx[1], sem_off[1])
        drain_out(0)
        compute_half(k2, 0, 0)
        issue_out(k2, 0)
        drain_out(1)
        compute_half(k2, 0, 1)
        issue_out(k2, 1)
        return carry

    lax.fori_loop(0, (K - 1) // 2, pair_body, 0)

    drain_out(0)
    drain_out(1)


_sc_call = functools.partial(
    pl.kernel,
    out_type=jax.ShapeDtypeStruct((N, C * K, 2, HALF), jnp.float32),
    mesh=plsc.VectorSubcoreMesh(core_axis_name="c", subcore_axis_name="s"),
    compiler_params=pltpu.CompilerParams(needs_layout_passes=False),
    scratch_types=(
        [pltpu.VMEM((HW,), jnp.int32) for _ in range(CP)]
        + [pltpu.VMEM((HW,), jnp.float32) for _ in range(CB)]
        + [pltpu.VMEM((HW,), jnp.float32) for _ in range(4)]
        + [pltpu.SemaphoreType.DMA for _ in range(5)]
    ),
)(_body)


@jax.jit
def kernel(target_offset, dcn_offset):
    tgt = target_offset.reshape(N, C, HW)
    dcn = dcn_offset.reshape(N, 2 * K, HW)
    out = _sc_call(tgt, dcn)
    return out.reshape(N, C * K, H, W)


# speedup vs baseline: 1.1039x; 1.1039x over previous
import functools

import jax
import jax.numpy as jnp
from jax import lax
from jax.experimental import pallas as pl
from jax.experimental.pallas import tpu as pltpu
from jax.experimental.pallas import tpu_sc as plsc

N, C, H, W = 4, 128, 64, 64
K = 9
HW = H * W
CB = 8
CP = CB // 2
NTASK = N * (C // CB)
NWORKERS = 32
TPW = NTASK // NWORKERS
LPP = HW // 16


def _body(tgt_hbm, dcn_hbm, out_hbm, *refs):
    pplanes = refs[0:CP]
    outs = (refs[CP:CP + CB], refs[CP + CB:CP + 2 * CB])
    base = CP + 2 * CB
    offy = (refs[base], refs[base + 1])
    offx = (refs[base + 2], refs[base + 3])
    tmp = refs[base + 4:base + 8]
    sem_pl = (refs[base + 8], refs[base + 9])
    sem_off = (refs[base + 10], refs[base + 11])
    sem_out = (refs[base + 12], refs[base + 13])

    wid = lax.axis_index("s") * 2 + lax.axis_index("c")
    lane = lax.iota(jnp.int32, 16)

    def compute_point(k, b, n, c0):
        ay = (k // 3 - 1).astype(jnp.float32)
        ax = (k % 3 - 1).astype(jnp.float32)
        offy_v = offy[b]
        offx_v = offx[b]
        outs_b = outs[b]

        @plsc.parallel_loop(0, LPP, unroll=1)
        def jbody(j):
            oy = offy_v[pl.ds(j * 16, 16)]
            ox = offx_v[pl.ds(j * 16, 16)]
            h = j // 4
            wb = (j % 4) * 16

            yv = oy + (h.astype(jnp.float32) + ay)
            xv = ox + (wb + lane).astype(jnp.float32) + ax

            ytf = yv.astype(jnp.int32).astype(jnp.float32)
            y0f = ytf - jnp.where(ytf > yv, 1.0, 0.0)
            xtf = xv.astype(jnp.int32).astype(jnp.float32)
            x0f = xtf - jnp.where(xtf > xv, 1.0, 0.0)
            y1f = y0f + 1.0
            x1f = x0f + 1.0

            wy1 = yv - y0f
            wy0 = 1.0 - wy1
            wx1 = xv - x0f
            wx0 = 1.0 - wx1
            y0c = jnp.minimum(jnp.maximum(y0f, 0.0), H - 1.0)
            y1c = jnp.minimum(jnp.maximum(y1f, 0.0), H - 1.0)
            x0c = jnp.minimum(jnp.maximum(x0f, 0.0), W - 1.0)
            x1c = jnp.minimum(jnp.maximum(x1f, 0.0), W - 1.0)
            wy0 = jnp.where(y0f == y0c, wy0, 0.0)
            wy1 = jnp.where(y1f == y1c, wy1, 0.0)
            wx0 = jnp.where(x0f == x0c, wx0, 0.0)
            wx1 = jnp.where(x1f == x1c, wx1, 0.0)

            y0i = y0c.astype(jnp.int32)
            y1i = y1c.astype(jnp.int32)
            x0i = x0c.astype(jnp.int32)
            x1i = x1c.astype(jnp.int32)

            iy0 = y0i * W
            iy1 = y1i * W
            i00 = iy0 + x0i
            i01 = iy0 + x1i
            i10 = iy1 + x0i
            i11 = iy1 + x1i

            wy0p = plsc.pack(wy0, wy0, format=plsc.PackFormat.INTERLEAVED)
            wy1p = plsc.pack(wy1, wy1, format=plsc.PackFormat.INTERLEAVED)
            wx0p = plsc.pack(wx0, wx0, format=plsc.PackFormat.INTERLEAVED)
            wx1p = plsc.pack(wx1, wx1, format=plsc.PackFormat.INTERLEAVED)

            for cp in range(CP):
                v00 = plsc.bitcast(
                    plsc.load_gather(pplanes[cp], [i00]), jnp.bfloat16)
                v01 = plsc.bitcast(
                    plsc.load_gather(pplanes[cp], [i01]), jnp.bfloat16)
                v10 = plsc.bitcast(
                    plsc.load_gather(pplanes[cp], [i10]), jnp.bfloat16)
                v11 = plsc.bitcast(
                    plsc.load_gather(pplanes[cp], [i11]), jnp.bfloat16)
                accp = (v00 * wx0p + v01 * wx1p) * wy0p \
                    + (v10 * wx0p + v11 * wx1p) * wy1p
                a0, a1 = plsc.unpack(accp, format=plsc.PackFormat.INTERLEAVED)
                outs_b[2 * cp][pl.ds(j * 16, 16)] = a0
                outs_b[2 * cp + 1][pl.ds(j * 16, 16)] = a1

    def drain_out(b, n):
        for ci in range(CB):
            pltpu.make_async_copy(
                outs[b][ci], out_hbm.at[n, ci], sem_out[b]).wait()

    def issue_out(k, b, n, c0):
        for ci in range(CB):
            pltpu.async_copy(
                outs[b][ci], out_hbm.at[n, (c0 + ci) * K + k], sem_out[b])

    def prefetch_off(k, b, n):
        pltpu.async_copy(dcn_hbm.at[n, 2 * k], offy[b], sem_off[b])
        pltpu.async_copy(dcn_hbm.at[n, 2 * k + 1], offx[b], sem_off[b])

    def drain_off(b, n):
        pltpu.make_async_copy(dcn_hbm.at[n, 0], offy[b], sem_off[b]).wait()
        pltpu.make_async_copy(dcn_hbm.at[n, 0], offx[b], sem_off[b]).wait()

    def issue_pair(cp, n, c0):
        pb = cp % 2
        pltpu.async_copy(tgt_hbm.at[n, c0 + 2 * cp], tmp[2 * pb], sem_pl[pb])
        pltpu.async_copy(
            tgt_hbm.at[n, c0 + 2 * cp + 1], tmp[2 * pb + 1], sem_pl[pb])

    def drain_pair(cp, n):
        pb = cp % 2
        pltpu.make_async_copy(
            tgt_hbm.at[n, 0], tmp[2 * pb], sem_pl[pb]).wait()
        pltpu.make_async_copy(
            tgt_hbm.at[n, 0], tmp[2 * pb + 1], sem_pl[pb]).wait()

    for tt in range(TPW):
        t = wid + NWORKERS * tt
        n = t // (C // CB)
        c0 = (t % (C // CB)) * CB

        issue_pair(0, n, c0)
        prefetch_off(0, 0, n)
        for cp in range(CP):
            if cp + 1 < CP:
                issue_pair(cp + 1, n, c0)
            drain_pair(cp, n)
            ta = tmp[2 * (cp % 2)]
            tb = tmp[2 * (cp % 2) + 1]
            pp = pplanes[cp]

            @plsc.parallel_loop(0, LPP, unroll=2)
            def pack_body(j, ta=ta, tb=tb, pp=pp):
                va = ta[pl.ds(j * 16, 16)]
                vb = tb[pl.ds(j * 16, 16)]
                packed = plsc.pack(va, vb, format=plsc.PackFormat.INTERLEAVED)
                pp[pl.ds(j * 16, 16)] = plsc.bitcast(packed, jnp.int32)

        k0 = jnp.int32(0)
        drain_off(0, n)
        prefetch_off(1, 1, n)
        if tt > 0:
            drain_out(0, n)
        compute_point(k0, 0, n, c0)
        issue_out(k0, 0, n, c0)

        def pair_body(kk, carry, tt=tt, n=n, c0=c0):
            k1 = 2 * kk + 1
            drain_off(1, n)
            pltpu.async_copy(dcn_hbm.at[n, 2 * (k1 + 1)], offy[0], sem_off[0])
            pltpu.async_copy(
                dcn_hbm.at[n, 2 * (k1 + 1) + 1], offx[0], sem_off[0])
            if tt > 0:
                drain_out(1, n)
            else:
                @pl.when(kk > 0)
                def _():
                    drain_out(1, n)
            compute_point(k1, 1, n, c0)
            issue_out(k1, 1, n, c0)

            k2 = k1 + 1
            drain_off(0, n)

            @pl.when(kk < 3)
            def _():
                pltpu.async_copy(
                    dcn_hbm.at[n, 2 * (k2 + 1)], offy[1], sem_off[1])
                pltpu.async_copy(
                    dcn_hbm.at[n, 2 * (k2 + 1) + 1], offx[1], sem_off[1])
            drain_out(0, n)
            compute_point(k2, 0, n, c0)
            issue_out(k2, 0, n, c0)
            return carry

        lax.fori_loop(0, (K - 1) // 2, pair_body, 0)

    drain_out(0, 0)
    drain_out(1, 0)


_sc_call = functools.partial(
    pl.kernel,
    out_type=jax.ShapeDtypeStruct((N, C * K, HW), jnp.float32),
    mesh=plsc.VectorSubcoreMesh(core_axis_name="c", subcore_axis_name="s"),
    compiler_params=pltpu.CompilerParams(needs_layout_passes=False),
    scratch_types=(
        [pltpu.VMEM((HW,), jnp.int32) for _ in range(CP)]
        + [pltpu.VMEM((HW,), jnp.float32) for _ in range(2 * CB)]
        + [pltpu.VMEM((HW,), jnp.float32) for _ in range(4)]
        + [pltpu.VMEM((HW,), jnp.float32) for _ in range(4)]
        + [pltpu.SemaphoreType.DMA for _ in range(6)]
    ),
)(_body)


@jax.jit
def kernel(target_offset, dcn_offset):
    tgt = target_offset.reshape(N, C, HW)
    dcn = dcn_offset.reshape(N, 2 * K, HW)
    out = _sc_call(tgt, dcn)
    return out.reshape(N, C * K, H, W)
